# Initial kernel scaffold; baseline (speedup 1.0000x reference)
import jax
import jax.numpy as jnp
from jax.experimental import pallas as pl


def kernel(x, NUM_NEIGHBORS, EDGE_METHOD, W1, b1, g1, be1, Wl, Wr, att, gb, W2, b2, g2, be2):
    raise NotImplementedError("write your pallas kernel here")


# speedup vs baseline: 14.4087x; 14.4087x over previous
import functools

import jax
import jax.numpy as jnp
from jax import lax
from jax.experimental import pallas as pl
from jax.experimental.pallas import tpu as pltpu
from jax.experimental.pallas import tpu_sc as plsc

B = 4096
C = 16
P = 16
HEADS = 4
D = C * P
DH = HEADS * D
K = 16
NWORK = 32
NPW = B // NWORK
NEG = -3.0e38


def _conv1_body(x_ref, m_ref, b_ref, h_ref, s_ref):
    h = jnp.dot(x_ref[...], m_ref[...], preferred_element_type=jnp.float32)
    h = h + b_ref[...]
    h_ref[...] = h
    s_ref[0:1, :] = jnp.sum(h, axis=0, keepdims=True)
    s_ref[1:2, :] = jnp.sum(h * h, axis=0, keepdims=True)


def _conv1(x2d, m1, bb1):
    return pl.pallas_call(
        _conv1_body,
        grid=(1,),
        in_specs=[
            pl.BlockSpec((B, D), lambda i: (0, 0)),
            pl.BlockSpec((D, D), lambda i: (0, 0)),
            pl.BlockSpec((1, D), lambda i: (0, 0)),
        ],
        out_specs=[
            pl.BlockSpec((B, D), lambda i: (0, 0)),
            pl.BlockSpec((2, D), lambda i: (0, 0)),
        ],
        out_shape=[
            jax.ShapeDtypeStruct((B, D), jnp.float32),
            jax.ShapeDtypeStruct((2, D), jnp.float32),
        ],
    )(x2d, m1, bb1)


def _proj_body(h_ref, g_ref, mu_ref, den_ref, be_ref, wl_ref, wr_ref,
               f_ref, xl_ref, xr_ref):
    f = (g_ref[...] * (h_ref[...] - mu_ref[...])) / den_ref[...] + be_ref[...]
    f_ref[...] = f
    xl_ref[...] = jnp.dot(f, wl_ref[...], preferred_element_type=jnp.float32)
    xr_ref[...] = jnp.dot(f, wr_ref[...], preferred_element_type=jnp.float32)


def _proj(h, gcol, mucol, dencol, becol, wl, wr):
    blk = B // 4
    return pl.pallas_call(
        _proj_body,
        grid=(4,),
        in_specs=[
            pl.BlockSpec((blk, D), lambda i: (i, 0)),
            pl.BlockSpec((1, D), lambda i: (0, 0)),
            pl.BlockSpec((1, D), lambda i: (0, 0)),
            pl.BlockSpec((1, D), lambda i: (0, 0)),
            pl.BlockSpec((1, D), lambda i: (0, 0)),
            pl.BlockSpec((D, DH), lambda i: (0, 0)),
            pl.BlockSpec((D, DH), lambda i: (0, 0)),
        ],
        out_specs=[
            pl.BlockSpec((blk, D), lambda i: (i, 0)),
            pl.BlockSpec((blk, DH), lambda i: (i, 0)),
            pl.BlockSpec((blk, DH), lambda i: (i, 0)),
        ],
        out_shape=[
            jax.ShapeDtypeStruct((B, D), jnp.float32),
            jax.ShapeDtypeStruct((B, DH), jnp.float32),
            jax.ShapeDtypeStruct((B, DH), jnp.float32),
        ],
    )(h, gcol, mucol, dencol, becol, wl, wr)


def _knn_body(f_ref, ft_ref, idx_ref, sq_scr):
    i = pl.program_id(0)

    @pl.when(i == 0)
    def _():
        ft = ft_ref[...]
        sq_scr[...] = jnp.sum(ft * ft, axis=0, keepdims=True)

    fb = f_ref[...]
    sq_b = jnp.sum(fb * fb, axis=1, keepdims=True)
    t = lax.dot_general(fb, ft_ref[...], (((1,), (0,)), ((), ())),
                        preferred_element_type=jnp.float32)
    d2 = (sq_b + sq_scr[...]) - 2.0 * t
    rblk = fb.shape[0]
    cols = lax.broadcasted_iota(jnp.int32, (rblk, B), 1)
    rows_g = i * rblk + lax.broadcasted_iota(jnp.int32, (rblk, B), 0)
    neg = jnp.where(cols == rows_g, NEG, -d2)
    for k in range(K):
        m = jnp.max(neg, axis=1, keepdims=True)
        cand = jnp.where(neg >= m, cols, B)
        amin = jnp.min(cand, axis=1, keepdims=True)
        idx_ref[:, k:k + 1] = amin
        neg = jnp.where(cols == amin, NEG, neg)


def _knn(feat, featT):
    blk = 256
    return pl.pallas_call(
        _knn_body,
        grid=(B // blk,),
        in_specs=[
            pl.BlockSpec((blk, D), lambda i: (i, 0)),
            pl.BlockSpec((D, B), lambda i: (0, 0)),
        ],
        out_specs=pl.BlockSpec((blk, K), lambda i: (i, 0)),
        out_shape=jax.ShapeDtypeStruct((B, K), jnp.int32),
        scratch_shapes=[pltpu.VMEM((1, B), jnp.float32)],
    )(feat, featT)


def _conv2_body(go_ref, gb_ref, m_ref, b_ref, t_ref, s_ref):
    g = go_ref[...] + gb_ref[...]
    t = jnp.dot(g, m_ref[...], preferred_element_type=jnp.float32)
    t = t + b_ref[...]
    t_ref[...] = t
    s_ref[0:1, :] = jnp.sum(t, axis=0, keepdims=True)
    s_ref[1:2, :] = jnp.sum(t * t, axis=0, keepdims=True)


def _conv2(go, gbrow, m2, bb2):
    return pl.pallas_call(
        _conv2_body,
        grid=(1,),
        in_specs=[
            pl.BlockSpec((B, DH), lambda i: (0, 0)),
            pl.BlockSpec((1, DH), lambda i: (0, 0)),
            pl.BlockSpec((DH, D), lambda i: (0, 0)),
            pl.BlockSpec((1, D), lambda i: (0, 0)),
        ],
        out_specs=[
            pl.BlockSpec((B, D), lambda i: (0, 0)),
            pl.BlockSpec((2, D), lambda i: (0, 0)),
        ],
        out_shape=[
            jax.ShapeDtypeStruct((B, D), jnp.float32),
            jax.ShapeDtypeStruct((2, D), jnp.float32),
        ],
    )(go, gbrow, m2, bb2)


def _bnres_body(t_ref, g_ref, mu_ref, den_ref, be_ref, x_ref, y_ref):
    y = (g_ref[...] * (t_ref[...] - mu_ref[...])) / den_ref[...] + be_ref[...]
    y_ref[...] = y + x_ref[...]


def _bnres(t2d, gcol, mucol, dencol, becol, x2d):
    return pl.pallas_call(
        _bnres_body,
        grid=(1,),
        in_specs=[
            pl.BlockSpec((B, D), lambda i: (0, 0)),
            pl.BlockSpec((1, D), lambda i: (0, 0)),
            pl.BlockSpec((1, D), lambda i: (0, 0)),
            pl.BlockSpec((1, D), lambda i: (0, 0)),
            pl.BlockSpec((1, D), lambda i: (0, 0)),
            pl.BlockSpec((B, D), lambda i: (0, 0)),
        ],
        out_specs=pl.BlockSpec((B, D), lambda i: (0, 0)),
        out_shape=jax.ShapeDtypeStruct((B, D), jnp.float32),
    )(t2d, gcol, mucol, dencol, becol, x2d)


def _bn_cols(stats, g, be, nch):
    n = jnp.float32(B * P)
    cs = stats[0].reshape(nch, P).sum(axis=1)
    css = stats[1].reshape(nch, P).sum(axis=1)
    mu = cs / n
    var = css / n - mu * mu
    den = jnp.sqrt(var + 1e-5)
    rep = lambda v: jnp.repeat(v, P).reshape(1, nch * P)
    return rep(g), rep(mu), rep(den), rep(be)


_GATHER_DNUMS = lax.GatherDimensionNumbers(
    offset_dims=(), collapsed_slice_dims=(0,), start_index_map=(0,))


def _take(v, idx):
    return lax.gather(v, idx[:, None], _GATHER_DNUMS, (1,),
                      mode=lax.GatherScatterMode.PROMISE_IN_BOUNDS)


def _bsum(v, lane):
    for s in (8, 4, 2, 1):
        v = v + _take(v, lane ^ s)
    return v


def _bmax(v, lane):
    for s in (8, 4, 2, 1):
        v = jnp.maximum(v, _take(v, lane ^ s))
    return v


def _logits_chunks(rows_v, xr_v, att_v, lane, kk):
    outs = []
    for h in range(HEADS):
        acc = jnp.zeros((16,), jnp.float32)
        for c in range(P):
            off = h * D + c * 16
            e = rows_v[kk, pl.ds(off, 16)] + xr_v[0, pl.ds(off, 16)]
            e = jnp.maximum(e, 0.2 * e)
            acc = acc + e * att_v[pl.ds(off, 16)]
        outs.append(_bsum(acc, lane))
    return outs


def _sc_agg_body(xl_hbm, xr_hbm, idx_hbm, att_hbm, go_hbm,
                 idxs_v, att_v, rows_v, xr_v, out_v, sem):
    wid = lax.axis_index("s") * 2 + lax.axis_index("c")
    base = wid * NPW
    pltpu.sync_copy(idx_hbm.at[pl.ds(base * K, NPW * K)], idxs_v)
    pltpu.sync_copy(att_hbm.at[:], att_v)
    lane = lax.iota(jnp.int32, 16)

    def node_body(n, carry):
        node = base + n
        cp = pltpu.make_async_copy(xl_hbm.at[idxs_v.at[pl.ds(n * K, K)]],
                                   rows_v.at[pl.ds(0, K)], sem)
        cp.start()
        pltpu.sync_copy(xl_hbm.at[pl.ds(node, 1)], rows_v.at[pl.ds(K, 1)])
        pltpu.sync_copy(xr_hbm.at[pl.ds(node, 1)], xr_v)
        cp.wait()

        def k_body(kk, lgs):
            sc = _logits_chunks(rows_v, xr_v, att_v, lane, kk)
            return tuple(jnp.where(lane == kk, sc[h], lgs[h])
                         for h in range(HEADS))

        zero4 = tuple(jnp.zeros((16,), jnp.float32) for _ in range(HEADS))
        lgs = lax.fori_loop(0, K, k_body, zero4)
        sv = _logits_chunks(rows_v, xr_v, att_v, lane, K)

        alvecs = []
        for h in range(HEADS):
            v0 = lgs[h]
            m = jnp.maximum(_bmax(v0, lane), sv[h])
            e0 = jnp.exp(v0 - m)
            es = jnp.exp(sv[h] - m)
            den = _bsum(e0, lane) + es + 1e-16
            alvecs.append((e0 / den, es / den))

        for h in range(HEADS):
            a0, aself = alvecs[h]
            alphas = [_take(a0, jnp.full((16,), kk, jnp.int32))
                      for kk in range(K)] + [aself]

            def c_body(cc, c2, h=h, alphas=alphas):
                off = pl.multiple_of(h * D + cc * 16, 16)
                acc = alphas[0] * rows_v[0, pl.ds(off, 16)]
                for kk in range(1, K + 1):
                    acc = acc + alphas[kk] * rows_v[kk, pl.ds(off, 16)]
                out_v[0, pl.ds(off, 16)] = acc
                return c2

            lax.fori_loop(0, P, c_body, 0)

        pltpu.sync_copy(out_v, go_hbm.at[pl.ds(node, 1)])
        return carry

    lax.fori_loop(0, NPW, node_body, 0)


def _sc_agg(xl, xr, idx, attf):
    mesh = plsc.VectorSubcoreMesh(core_axis_name="c", subcore_axis_name="s")
    return pl.kernel(
        _sc_agg_body,
        out_type=jax.ShapeDtypeStruct((B, DH), jnp.float32),
        mesh=mesh,
        scratch_types=[
            pltpu.VMEM((NPW * K,), jnp.int32),
            pltpu.VMEM((DH,), jnp.float32),
            pltpu.VMEM((K + 1, DH), jnp.float32),
            pltpu.VMEM((1, DH), jnp.float32),
            pltpu.VMEM((1, DH), jnp.float32),
            pltpu.SemaphoreType.DMA,
        ],
    )(xl, xr, idx.reshape(B * K), attf)


def kernel(x, NUM_NEIGHBORS, EDGE_METHOD, W1, b1, g1, be1, Wl, Wr, att, gb,
           W2, b2, g2, be2):
    x2d = x.reshape(B, D)
    eye = jnp.eye(P, dtype=jnp.float32)
    m1 = jnp.kron(W1.T, eye)
    bb1 = jnp.repeat(b1, P).reshape(1, D)
    m2 = jnp.kron(W2.T, eye)
    bb2 = jnp.repeat(b2, P).reshape(1, D)
    attf = att.reshape(DH)
    gbrow = gb.reshape(1, DH)

    h, s1 = _conv1(x2d, m1, bb1)
    g1c, mu1c, den1c, be1c = _bn_cols(s1, g1, be1, C)
    feat, xl, xr = _proj(h, g1c, mu1c, den1c, be1c, Wl, Wr)
    idx = _knn(feat, feat.T)
    go = _sc_agg(xl, xr, idx, attf)
    t2d, s2 = _conv2(go, gbrow, m2, bb2)
    g2c, mu2c, den2c, be2c = _bn_cols(s2, g2, be2, C)
    y2d = _bnres(t2d, g2c, mu2c, den2c, be2c, x2d)
    return y2d.reshape(B, C, 4, 4)


# speedup vs baseline: 19.6963x; 1.3670x over previous
import functools

import jax
import jax.numpy as jnp
from jax import lax
from jax.experimental import pallas as pl
from jax.experimental.pallas import tpu as pltpu
from jax.experimental.pallas import tpu_sc as plsc

B = 4096
C = 16
P = 16
HEADS = 4
D = C * P
DH = HEADS * D
K = 16
NWORK = 32
NPW = B // NWORK
NEG = -3.0e38


def _conv1_body(x_ref, m_ref, b_ref, h_ref, s_ref):
    h = jnp.dot(x_ref[...], m_ref[...], preferred_element_type=jnp.float32)
    h = h + b_ref[...]
    h_ref[...] = h
    s_ref[0:1, :] = jnp.sum(h, axis=0, keepdims=True)
    s_ref[1:2, :] = jnp.sum(h * h, axis=0, keepdims=True)


def _conv1(x2d, m1, bb1):
    return pl.pallas_call(
        _conv1_body,
        grid=(1,),
        in_specs=[
            pl.BlockSpec((B, D), lambda i: (0, 0)),
            pl.BlockSpec((D, D), lambda i: (0, 0)),
            pl.BlockSpec((1, D), lambda i: (0, 0)),
        ],
        out_specs=[
            pl.BlockSpec((B, D), lambda i: (0, 0)),
            pl.BlockSpec((2, D), lambda i: (0, 0)),
        ],
        out_shape=[
            jax.ShapeDtypeStruct((B, D), jnp.float32),
            jax.ShapeDtypeStruct((2, D), jnp.float32),
        ],
    )(x2d, m1, bb1)


def _proj_body(h_ref, g_ref, mu_ref, den_ref, be_ref, wl_ref, wr_ref,
               f_ref, xl_ref, xr_ref):
    f = (g_ref[...] * (h_ref[...] - mu_ref[...])) / den_ref[...] + be_ref[...]
    f_ref[...] = f
    xl_ref[...] = jnp.dot(f, wl_ref[...], preferred_element_type=jnp.float32)
    xr_ref[...] = jnp.dot(f, wr_ref[...], preferred_element_type=jnp.float32)


def _proj(h, gcol, mucol, dencol, becol, wl, wr):
    blk = B // 4
    return pl.pallas_call(
        _proj_body,
        grid=(4,),
        in_specs=[
            pl.BlockSpec((blk, D), lambda i: (i, 0)),
            pl.BlockSpec((1, D), lambda i: (0, 0)),
            pl.BlockSpec((1, D), lambda i: (0, 0)),
            pl.BlockSpec((1, D), lambda i: (0, 0)),
            pl.BlockSpec((1, D), lambda i: (0, 0)),
            pl.BlockSpec((D, DH), lambda i: (0, 0)),
            pl.BlockSpec((D, DH), lambda i: (0, 0)),
        ],
        out_specs=[
            pl.BlockSpec((blk, D), lambda i: (i, 0)),
            pl.BlockSpec((blk, DH), lambda i: (i, 0)),
            pl.BlockSpec((blk, DH), lambda i: (i, 0)),
        ],
        out_shape=[
            jax.ShapeDtypeStruct((B, D), jnp.float32),
            jax.ShapeDtypeStruct((B, DH), jnp.float32),
            jax.ShapeDtypeStruct((B, DH), jnp.float32),
        ],
    )(h, gcol, mucol, dencol, becol, wl, wr)


def _knn_body(f_ref, ft_ref, idx_ref, sq_scr):
    i = pl.program_id(0)

    @pl.when(i == 0)
    def _():
        ft = ft_ref[...]
        sq_scr[...] = jnp.sum(ft * ft, axis=0, keepdims=True)

    fb = f_ref[...]
    sq_b = jnp.sum(fb * fb, axis=1, keepdims=True)
    t = lax.dot_general(fb, ft_ref[...], (((1,), (0,)), ((), ())),
                        preferred_element_type=jnp.float32)
    d2 = (sq_b + sq_scr[...]) - 2.0 * t
    rblk = fb.shape[0]
    cols = lax.broadcasted_iota(jnp.int32, (rblk, B), 1)
    rows_g = i * rblk + lax.broadcasted_iota(jnp.int32, (rblk, B), 0)
    neg = jnp.where(cols == rows_g, NEG, -d2)
    for k in range(K):
        m = jnp.max(neg, axis=1, keepdims=True)
        cand = jnp.where(neg >= m, cols, B)
        amin = jnp.min(cand, axis=1, keepdims=True)
        idx_ref[:, k:k + 1] = amin
        neg = jnp.where(cols == amin, NEG, neg)


def _knn(feat, featT):
    blk = 256
    return pl.pallas_call(
        _knn_body,
        grid=(B // blk,),
        in_specs=[
            pl.BlockSpec((blk, D), lambda i: (i, 0)),
            pl.BlockSpec((D, B), lambda i: (0, 0)),
        ],
        out_specs=pl.BlockSpec((blk, K), lambda i: (i, 0)),
        out_shape=jax.ShapeDtypeStruct((B, K), jnp.int32),
        scratch_shapes=[pltpu.VMEM((1, B), jnp.float32)],
    )(feat, featT)


def _conv2_body(go_ref, gb_ref, m_ref, b_ref, t_ref, s_ref):
    g = go_ref[...] + gb_ref[...]
    t = jnp.dot(g, m_ref[...], preferred_element_type=jnp.float32)
    t = t + b_ref[...]
    t_ref[...] = t
    s_ref[0:1, :] = jnp.sum(t, axis=0, keepdims=True)
    s_ref[1:2, :] = jnp.sum(t * t, axis=0, keepdims=True)


def _conv2(go, gbrow, m2, bb2):
    return pl.pallas_call(
        _conv2_body,
        grid=(1,),
        in_specs=[
            pl.BlockSpec((B, DH), lambda i: (0, 0)),
            pl.BlockSpec((1, DH), lambda i: (0, 0)),
            pl.BlockSpec((DH, D), lambda i: (0, 0)),
            pl.BlockSpec((1, D), lambda i: (0, 0)),
        ],
        out_specs=[
            pl.BlockSpec((B, D), lambda i: (0, 0)),
            pl.BlockSpec((2, D), lambda i: (0, 0)),
        ],
        out_shape=[
            jax.ShapeDtypeStruct((B, D), jnp.float32),
            jax.ShapeDtypeStruct((2, D), jnp.float32),
        ],
    )(go, gbrow, m2, bb2)


def _bnres_body(t_ref, g_ref, mu_ref, den_ref, be_ref, x_ref, y_ref):
    y = (g_ref[...] * (t_ref[...] - mu_ref[...])) / den_ref[...] + be_ref[...]
    y_ref[...] = y + x_ref[...]


def _bnres(t2d, gcol, mucol, dencol, becol, x2d):
    return pl.pallas_call(
        _bnres_body,
        grid=(1,),
        in_specs=[
            pl.BlockSpec((B, D), lambda i: (0, 0)),
            pl.BlockSpec((1, D), lambda i: (0, 0)),
            pl.BlockSpec((1, D), lambda i: (0, 0)),
            pl.BlockSpec((1, D), lambda i: (0, 0)),
            pl.BlockSpec((1, D), lambda i: (0, 0)),
            pl.BlockSpec((B, D), lambda i: (0, 0)),
        ],
        out_specs=pl.BlockSpec((B, D), lambda i: (0, 0)),
        out_shape=jax.ShapeDtypeStruct((B, D), jnp.float32),
    )(t2d, gcol, mucol, dencol, becol, x2d)


def _bn_cols(stats, g, be, nch):
    n = jnp.float32(B * P)
    cs = stats[0].reshape(nch, P).sum(axis=1)
    css = stats[1].reshape(nch, P).sum(axis=1)
    mu = cs / n
    var = css / n - mu * mu
    den = jnp.sqrt(var + 1e-5)
    rep = lambda v: jnp.repeat(v, P).reshape(1, nch * P)
    return rep(g), rep(mu), rep(den), rep(be)


_GATHER_DNUMS = lax.GatherDimensionNumbers(
    offset_dims=(), collapsed_slice_dims=(0,), start_index_map=(0,))


def _take(v, idx):
    return lax.gather(v, idx[:, None], _GATHER_DNUMS, (1,),
                      mode=lax.GatherScatterMode.PROMISE_IN_BOUNDS)


def _bsum(v, lane):
    for s in (8, 4, 2, 1):
        v = v + _take(v, lane ^ s)
    return v


def _bmax(v, lane):
    for s in (8, 4, 2, 1):
        v = jnp.maximum(v, _take(v, lane ^ s))
    return v


def _sc_agg_body(xl_hbm, xr_hbm, idx_hbm, att_hbm, go_hbm,
                 idxs_v, att_v, rows_a, rows_b, xr_a, xr_b, out_a, out_b,
                 sem_a, sem_b, osem_a, osem_b):
    wid = lax.axis_index("s") * 2 + lax.axis_index("c")
    base = wid * NPW
    pltpu.sync_copy(idx_hbm.at[pl.ds(base * K, NPW * K)], idxs_v)
    pltpu.sync_copy(att_hbm.at[:], att_v)
    lane = lax.iota(jnp.int32, 16)

    def launch(n, rows_v, xr_v, sem):
        node = base + n
        pltpu.make_async_copy(xl_hbm.at[idxs_v.at[pl.ds(n * K, K)]],
                              rows_v.at[pl.ds(0, K)], sem).start()
        pltpu.make_async_copy(xl_hbm.at[pl.ds(node, 1)],
                              rows_v.at[pl.ds(K, 1)], sem).start()
        pltpu.make_async_copy(xr_hbm.at[pl.ds(node, 1)], xr_v, sem).start()

    def wait_inputs(n, rows_v, xr_v, sem):
        pltpu.make_async_copy(xl_hbm.at[idxs_v.at[pl.ds(n * K, K)]],
                              rows_v.at[pl.ds(0, K)], sem).wait()
        pltpu.make_async_copy(xl_hbm.at[pl.ds(base, 1)],
                              rows_v.at[pl.ds(K, 1)], sem).wait()
        pltpu.make_async_copy(xr_hbm.at[pl.ds(base, 1)], xr_v, sem).wait()

    def compute(g, n, rows_v, xr_v, out_v, sem, osem):
        wait_inputs(n, rows_v, xr_v, sem)

        @pl.when(g > 0)
        def _():
            pltpu.make_async_copy(out_v, go_hbm.at[pl.ds(base, 1)],
                                  osem).wait()

        for h in range(HEADS):
            def c_body(c, accs, h=h):
                off = h * D + c * 16
                x = xr_v[0, pl.ds(off, 16)]
                a = att_v[pl.ds(off, 16)]
                new = []
                for kk in range(K + 1):
                    e = rows_v[kk, pl.ds(off, 16)] + x
                    e = jnp.maximum(e, 0.2 * e)
                    new.append(accs[kk] + e * a)
                return tuple(new)

            zeros = tuple(jnp.zeros((16,), jnp.float32)
                          for _ in range(K + 1))
            accs = lax.fori_loop(0, P, c_body, zeros)
            lg = [_bsum(v, lane) for v in accs]
            m = lg[0]
            for v in lg[1:]:
                m = jnp.maximum(m, v)
            es = [jnp.exp(v - m) for v in lg]
            den = es[0]
            for v in es[1:]:
                den = den + v
            den = den + 1e-16
            alph = [v / den for v in es]

            def w_body(c, c2, h=h, alph=alph):
                off = h * D + c * 16
                acc = alph[0] * rows_v[0, pl.ds(off, 16)]
                for kk in range(1, K + 1):
                    acc = acc + alph[kk] * rows_v[kk, pl.ds(off, 16)]
                out_v[0, pl.ds(off, 16)] = acc
                return c2

            lax.fori_loop(0, P, w_body, 0)

        node = base + n
        pltpu.make_async_copy(out_v, go_hbm.at[pl.ds(node, 1)], osem).start()

    launch(0, rows_a, xr_a, sem_a)
    launch(1, rows_b, xr_b, sem_b)

    def g_body(g, carry):
        compute(g, 2 * g, rows_a, xr_a, out_a, sem_a, osem_a)

        @pl.when(g < NPW // 2 - 1)
        def _():
            launch(2 * g + 2, rows_a, xr_a, sem_a)

        compute(g, 2 * g + 1, rows_b, xr_b, out_b, sem_b, osem_b)

        @pl.when(g < NPW // 2 - 1)
        def _():
            launch(2 * g + 3, rows_b, xr_b, sem_b)

        return carry

    lax.fori_loop(0, NPW // 2, g_body, 0)
    pltpu.make_async_copy(out_a, go_hbm.at[pl.ds(base, 1)], osem_a).wait()
    pltpu.make_async_copy(out_b, go_hbm.at[pl.ds(base, 1)], osem_b).wait()


def _sc_agg(xl, xr, idx, attf):
    mesh = plsc.VectorSubcoreMesh(core_axis_name="c", subcore_axis_name="s")
    return pl.kernel(
        _sc_agg_body,
        out_type=jax.ShapeDtypeStruct((B, DH), jnp.float32),
        mesh=mesh,
        scratch_types=[
            pltpu.VMEM((NPW * K,), jnp.int32),
            pltpu.VMEM((DH,), jnp.float32),
            pltpu.VMEM((K + 1, DH), jnp.float32),
            pltpu.VMEM((K + 1, DH), jnp.float32),
            pltpu.VMEM((1, DH), jnp.float32),
            pltpu.VMEM((1, DH), jnp.float32),
            pltpu.VMEM((1, DH), jnp.float32),
            pltpu.VMEM((1, DH), jnp.float32),
            pltpu.SemaphoreType.DMA,
            pltpu.SemaphoreType.DMA,
            pltpu.SemaphoreType.DMA,
            pltpu.SemaphoreType.DMA,
        ],
    )(xl, xr, idx.reshape(B * K), attf)


def kernel(x, NUM_NEIGHBORS, EDGE_METHOD, W1, b1, g1, be1, Wl, Wr, att, gb,
           W2, b2, g2, be2):
    x2d = x.reshape(B, D)
    eye = jnp.eye(P, dtype=jnp.float32)
    m1 = jnp.kron(W1.T, eye)
    bb1 = jnp.repeat(b1, P).reshape(1, D)
    m2 = jnp.kron(W2.T, eye)
    bb2 = jnp.repeat(b2, P).reshape(1, D)
    attf = att.reshape(DH)
    gbrow = gb.reshape(1, DH)

    h, s1 = _conv1(x2d, m1, bb1)
    g1c, mu1c, den1c, be1c = _bn_cols(s1, g1, be1, C)
    feat, xl, xr = _proj(h, g1c, mu1c, den1c, be1c, Wl, Wr)
    idx = _knn(feat, feat.T)
    go = _sc_agg(xl, xr, idx, attf)
    t2d, s2 = _conv2(go, gbrow, m2, bb2)
    g2c, mu2c, den2c, be2c = _bn_cols(s2, g2, be2, C)
    y2d = _bnres(t2d, g2c, mu2c, den2c, be2c, x2d)
    return y2d.reshape(B, C, 4, 4)


# speedup vs baseline: 23.1024x; 1.1729x over previous
import functools

import jax
import jax.numpy as jnp
from jax import lax
from jax.experimental import pallas as pl
from jax.experimental.pallas import tpu as pltpu
from jax.experimental.pallas import tpu_sc as plsc

B = 4096
C = 16
P = 16
HEADS = 4
D = C * P
DH = HEADS * D
K = 16
NWORK = 32
NPW = B // NWORK
NEG = -3.0e38


def _conv1_body(x_ref, m_ref, b_ref, h_ref, s_ref):
    h = jnp.dot(x_ref[...], m_ref[...], preferred_element_type=jnp.float32)
    h = h + b_ref[...]
    h_ref[...] = h
    s_ref[0:1, :] = jnp.sum(h, axis=0, keepdims=True)
    s_ref[1:2, :] = jnp.sum(h * h, axis=0, keepdims=True)


def _conv1(x2d, m1, bb1):
    return pl.pallas_call(
        _conv1_body,
        grid=(1,),
        in_specs=[
            pl.BlockSpec((B, D), lambda i: (0, 0)),
            pl.BlockSpec((D, D), lambda i: (0, 0)),
            pl.BlockSpec((1, D), lambda i: (0, 0)),
        ],
        out_specs=[
            pl.BlockSpec((B, D), lambda i: (0, 0)),
            pl.BlockSpec((2, D), lambda i: (0, 0)),
        ],
        out_shape=[
            jax.ShapeDtypeStruct((B, D), jnp.float32),
            jax.ShapeDtypeStruct((2, D), jnp.float32),
        ],
    )(x2d, m1, bb1)


def _proj_body(h_ref, g_ref, mu_ref, den_ref, be_ref, wl_ref, wr_ref,
               f_ref, xl_ref, xr_ref):
    f = (g_ref[...] * (h_ref[...] - mu_ref[...])) / den_ref[...] + be_ref[...]
    f_ref[...] = f
    xl_ref[...] = jnp.dot(f, wl_ref[...], preferred_element_type=jnp.float32)
    xr_ref[...] = jnp.dot(f, wr_ref[...], preferred_element_type=jnp.float32)


def _proj(h, gcol, mucol, dencol, becol, wl, wr):
    blk = B // 4
    return pl.pallas_call(
        _proj_body,
        grid=(4,),
        in_specs=[
            pl.BlockSpec((blk, D), lambda i: (i, 0)),
            pl.BlockSpec((1, D), lambda i: (0, 0)),
            pl.BlockSpec((1, D), lambda i: (0, 0)),
            pl.BlockSpec((1, D), lambda i: (0, 0)),
            pl.BlockSpec((1, D), lambda i: (0, 0)),
            pl.BlockSpec((D, DH), lambda i: (0, 0)),
            pl.BlockSpec((D, DH), lambda i: (0, 0)),
        ],
        out_specs=[
            pl.BlockSpec((blk, D), lambda i: (i, 0)),
            pl.BlockSpec((blk, DH), lambda i: (i, 0)),
            pl.BlockSpec((blk, DH), lambda i: (i, 0)),
        ],
        out_shape=[
            jax.ShapeDtypeStruct((B, D), jnp.float32),
            jax.ShapeDtypeStruct((B, DH), jnp.float32),
            jax.ShapeDtypeStruct((B, DH), jnp.float32),
        ],
    )(h, gcol, mucol, dencol, becol, wl, wr)


def _knn_body(f_ref, ft_ref, idx_ref, sq_scr, *, row0):
    i = pl.program_id(0)

    @pl.when(i == 0)
    def _():
        ft = ft_ref[...]
        sq_scr[...] = jnp.sum(ft * ft, axis=0, keepdims=True)

    fb = f_ref[...]
    sq_b = jnp.sum(fb * fb, axis=1, keepdims=True)
    t = lax.dot_general(fb, ft_ref[...], (((1,), (0,)), ((), ())),
                        preferred_element_type=jnp.float32)
    d2 = (sq_b + sq_scr[...]) - 2.0 * t
    rblk = fb.shape[0]
    cols = lax.broadcasted_iota(jnp.int32, (rblk, B), 1)
    rows_g = row0 + i * rblk + lax.broadcasted_iota(jnp.int32, (rblk, B), 0)
    neg = jnp.where(cols == rows_g, NEG, -d2)
    for k in range(K):
        m = jnp.max(neg, axis=1, keepdims=True)
        cand = jnp.where(neg >= m, cols, B)
        amin = jnp.min(cand, axis=1, keepdims=True)
        idx_ref[:, k:k + 1] = amin
        neg = jnp.where(cols == amin, NEG, neg)


def _knn(feat, featT, off, nrows):
    blk = 256
    ob = off // blk
    return pl.pallas_call(
        functools.partial(_knn_body, row0=off),
        grid=(nrows // blk,),
        in_specs=[
            pl.BlockSpec((blk, D), lambda i: (i + ob, 0)),
            pl.BlockSpec((D, B), lambda i: (0, 0)),
        ],
        out_specs=pl.BlockSpec((blk, K), lambda i: (i, 0)),
        out_shape=jax.ShapeDtypeStruct((nrows, K), jnp.int32),
        scratch_shapes=[pltpu.VMEM((1, B), jnp.float32)],
    )(feat, featT)


def _conv2_body(go_ref, gb_ref, m_ref, b_ref, t_ref, s_ref):
    g = go_ref[...] + gb_ref[...]
    t = jnp.dot(g, m_ref[...], preferred_element_type=jnp.float32)
    t = t + b_ref[...]
    t_ref[...] = t
    s_ref[0:1, :] = jnp.sum(t, axis=0, keepdims=True)
    s_ref[1:2, :] = jnp.sum(t * t, axis=0, keepdims=True)


def _conv2(go, gbrow, m2, bb2):
    return pl.pallas_call(
        _conv2_body,
        grid=(1,),
        in_specs=[
            pl.BlockSpec((B, DH), lambda i: (0, 0)),
            pl.BlockSpec((1, DH), lambda i: (0, 0)),
            pl.BlockSpec((DH, D), lambda i: (0, 0)),
            pl.BlockSpec((1, D), lambda i: (0, 0)),
        ],
        out_specs=[
            pl.BlockSpec((B, D), lambda i: (0, 0)),
            pl.BlockSpec((2, D), lambda i: (0, 0)),
        ],
        out_shape=[
            jax.ShapeDtypeStruct((B, D), jnp.float32),
            jax.ShapeDtypeStruct((2, D), jnp.float32),
        ],
    )(go, gbrow, m2, bb2)


def _bnres_body(t_ref, g_ref, mu_ref, den_ref, be_ref, x_ref, y_ref):
    y = (g_ref[...] * (t_ref[...] - mu_ref[...])) / den_ref[...] + be_ref[...]
    y_ref[...] = y + x_ref[...]


def _bnres(t2d, gcol, mucol, dencol, becol, x2d):
    return pl.pallas_call(
        _bnres_body,
        grid=(1,),
        in_specs=[
            pl.BlockSpec((B, D), lambda i: (0, 0)),
            pl.BlockSpec((1, D), lambda i: (0, 0)),
            pl.BlockSpec((1, D), lambda i: (0, 0)),
            pl.BlockSpec((1, D), lambda i: (0, 0)),
            pl.BlockSpec((1, D), lambda i: (0, 0)),
            pl.BlockSpec((B, D), lambda i: (0, 0)),
        ],
        out_specs=pl.BlockSpec((B, D), lambda i: (0, 0)),
        out_shape=jax.ShapeDtypeStruct((B, D), jnp.float32),
    )(t2d, gcol, mucol, dencol, becol, x2d)


def _bn_cols(stats, g, be, nch):
    n = jnp.float32(B * P)
    cs = stats[0].reshape(nch, P).sum(axis=1)
    css = stats[1].reshape(nch, P).sum(axis=1)
    mu = cs / n
    var = css / n - mu * mu
    den = jnp.sqrt(var + 1e-5)
    rep = lambda v: jnp.repeat(v, P).reshape(1, nch * P)
    return rep(g), rep(mu), rep(den), rep(be)


_GATHER_DNUMS = lax.GatherDimensionNumbers(
    offset_dims=(), collapsed_slice_dims=(0,), start_index_map=(0,))


def _take(v, idx):
    return lax.gather(v, idx[:, None], _GATHER_DNUMS, (1,),
                      mode=lax.GatherScatterMode.PROMISE_IN_BOUNDS)


def _bsum(v, lane):
    for s in (8, 4, 2, 1):
        v = v + _take(v, lane ^ s)
    return v


def _bmax(v, lane):
    for s in (8, 4, 2, 1):
        v = jnp.maximum(v, _take(v, lane ^ s))
    return v


def _sc_agg_body(xl_hbm, xr_hbm, idx_hbm, att_hbm, go_hbm,
                 idxs_v, att_v, rows_a, rows_b, xr_a, xr_b, out_a, out_b,
                 sem_a, sem_b, osem_a, osem_b, *, row_off, npw):
    wid = lax.axis_index("s") * 2 + lax.axis_index("c")
    base = wid * npw
    pltpu.sync_copy(idx_hbm.at[pl.ds(base * K, npw * K)], idxs_v)
    pltpu.sync_copy(att_hbm.at[:], att_v)
    lane = lax.iota(jnp.int32, 16)

    def launch(n, rows_v, xr_v, sem):
        node = row_off + base + n
        pltpu.make_async_copy(xl_hbm.at[idxs_v.at[pl.ds(n * K, K)]],
                              rows_v.at[pl.ds(0, K)], sem).start()
        pltpu.make_async_copy(xl_hbm.at[pl.ds(node, 1)],
                              rows_v.at[pl.ds(K, 1)], sem).start()
        pltpu.make_async_copy(xr_hbm.at[pl.ds(node, 1)], xr_v, sem).start()

    def wait_inputs(n, rows_v, xr_v, sem):
        pltpu.make_async_copy(xl_hbm.at[idxs_v.at[pl.ds(n * K, K)]],
                              rows_v.at[pl.ds(0, K)], sem).wait()
        pltpu.make_async_copy(xl_hbm.at[pl.ds(base, 1)],
                              rows_v.at[pl.ds(K, 1)], sem).wait()
        pltpu.make_async_copy(xr_hbm.at[pl.ds(base, 1)], xr_v, sem).wait()

    def compute(g, n, rows_v, xr_v, out_v, sem, osem):
        wait_inputs(n, rows_v, xr_v, sem)

        @pl.when(g > 0)
        def _():
            pltpu.make_async_copy(out_v, go_hbm.at[pl.ds(base, 1)],
                                  osem).wait()

        for h in range(HEADS):
            def c_body(c, accs, h=h):
                off = h * D + c * 16
                x = xr_v[0, pl.ds(off, 16)]
                a = att_v[pl.ds(off, 16)]
                new = []
                for kk in range(K + 1):
                    e = rows_v[kk, pl.ds(off, 16)] + x
                    e = jnp.maximum(e, 0.2 * e)
                    new.append(accs[kk] + e * a)
                return tuple(new)

            zeros = tuple(jnp.zeros((16,), jnp.float32)
                          for _ in range(K + 1))
            accs = lax.fori_loop(0, P, c_body, zeros)
            lg = [_bsum(v, lane) for v in accs]
            m = lg[0]
            for v in lg[1:]:
                m = jnp.maximum(m, v)
            es = [jnp.exp(v - m) for v in lg]
            den = es[0]
            for v in es[1:]:
                den = den + v
            den = den + 1e-16
            alph = [v / den for v in es]

            def w_body(c, c2, h=h, alph=alph):
                off = h * D + c * 16
                acc = alph[0] * rows_v[0, pl.ds(off, 16)]
                for kk in range(1, K + 1):
                    acc = acc + alph[kk] * rows_v[kk, pl.ds(off, 16)]
                out_v[0, pl.ds(off, 16)] = acc
                return c2

            lax.fori_loop(0, P, w_body, 0)

        node = base + n
        pltpu.make_async_copy(out_v, go_hbm.at[pl.ds(node, 1)], osem).start()

    launch(0, rows_a, xr_a, sem_a)
    launch(1, rows_b, xr_b, sem_b)

    def g_body(g, carry):
        compute(g, 2 * g, rows_a, xr_a, out_a, sem_a, osem_a)

        @pl.when(g < npw // 2 - 1)
        def _():
            launch(2 * g + 2, rows_a, xr_a, sem_a)

        compute(g, 2 * g + 1, rows_b, xr_b, out_b, sem_b, osem_b)

        @pl.when(g < npw // 2 - 1)
        def _():
            launch(2 * g + 3, rows_b, xr_b, sem_b)

        return carry

    lax.fori_loop(0, npw // 2, g_body, 0)
    pltpu.make_async_copy(out_a, go_hbm.at[pl.ds(base, 1)], osem_a).wait()
    pltpu.make_async_copy(out_b, go_hbm.at[pl.ds(base, 1)], osem_b).wait()


def _sc_agg(xl, xr, idx, attf, row_off, nrows):
    npw = nrows // NWORK
    mesh = plsc.VectorSubcoreMesh(core_axis_name="c", subcore_axis_name="s")
    return pl.kernel(
        functools.partial(_sc_agg_body, row_off=row_off, npw=npw),
        out_type=jax.ShapeDtypeStruct((nrows, DH), jnp.float32),
        mesh=mesh,
        scratch_types=[
            pltpu.VMEM((npw * K,), jnp.int32),
            pltpu.VMEM((DH,), jnp.float32),
            pltpu.VMEM((K + 1, DH), jnp.float32),
            pltpu.VMEM((K + 1, DH), jnp.float32),
            pltpu.VMEM((1, DH), jnp.float32),
            pltpu.VMEM((1, DH), jnp.float32),
            pltpu.VMEM((1, DH), jnp.float32),
            pltpu.VMEM((1, DH), jnp.float32),
            pltpu.SemaphoreType.DMA,
            pltpu.SemaphoreType.DMA,
            pltpu.SemaphoreType.DMA,
            pltpu.SemaphoreType.DMA,
        ],
    )(xl, xr, idx.reshape(nrows * K), attf)


def kernel(x, NUM_NEIGHBORS, EDGE_METHOD, W1, b1, g1, be1, Wl, Wr, att, gb,
           W2, b2, g2, be2):
    x2d = x.reshape(B, D)
    eye = jnp.eye(P, dtype=jnp.float32)
    m1 = jnp.kron(W1.T, eye)
    bb1 = jnp.repeat(b1, P).reshape(1, D)
    m2 = jnp.kron(W2.T, eye)
    bb2 = jnp.repeat(b2, P).reshape(1, D)
    attf = att.reshape(DH)
    gbrow = gb.reshape(1, DH)

    h, s1 = _conv1(x2d, m1, bb1)
    g1c, mu1c, den1c, be1c = _bn_cols(s1, g1, be1, C)
    feat, xl, xr = _proj(h, g1c, mu1c, den1c, be1c, Wl, Wr)
    featT = feat.T
    idx_a = _knn(feat, featT, 0, B // 2)
    idx_b = _knn(feat, featT, B // 2, B // 2)
    go_a = _sc_agg(xl, xr, idx_a, attf, 0, B // 2)
    go_b = _sc_agg(xl, xr, idx_b, attf, B // 2, B // 2)
    go = jnp.concatenate([go_a, go_b], axis=0)
    t2d, s2 = _conv2(go, gbrow, m2, bb2)
    g2c, mu2c, den2c, be2c = _bn_cols(s2, g2, be2, C)
    y2d = _bnres(t2d, g2c, mu2c, den2c, be2c, x2d)
    return y2d.reshape(B, C, 4, 4)


# speedup vs baseline: 24.2173x; 1.0483x over previous
import functools

import jax
import jax.numpy as jnp
from jax import lax
from jax.experimental import pallas as pl
from jax.experimental.pallas import tpu as pltpu
from jax.experimental.pallas import tpu_sc as plsc

B = 4096
C = 16
P = 16
HEADS = 4
D = C * P
DH = HEADS * D
K = 16
NWORK = 32
NPW = B // NWORK
NEG = -3.0e38


def _conv1_body(x_ref, m_ref, b_ref, h_ref, s_ref):
    h = jnp.dot(x_ref[...], m_ref[...], preferred_element_type=jnp.float32)
    h = h + b_ref[...]
    h_ref[...] = h
    s_ref[0:1, :] = jnp.sum(h, axis=0, keepdims=True)
    s_ref[1:2, :] = jnp.sum(h * h, axis=0, keepdims=True)


def _conv1(x2d, m1, bb1):
    return pl.pallas_call(
        _conv1_body,
        grid=(1,),
        in_specs=[
            pl.BlockSpec((B, D), lambda i: (0, 0)),
            pl.BlockSpec((D, D), lambda i: (0, 0)),
            pl.BlockSpec((1, D), lambda i: (0, 0)),
        ],
        out_specs=[
            pl.BlockSpec((B, D), lambda i: (0, 0)),
            pl.BlockSpec((2, D), lambda i: (0, 0)),
        ],
        out_shape=[
            jax.ShapeDtypeStruct((B, D), jnp.float32),
            jax.ShapeDtypeStruct((2, D), jnp.float32),
        ],
    )(x2d, m1, bb1)


def _proj_body(h_ref, g_ref, mu_ref, den_ref, be_ref, wl_ref, wr_ref,
               f_ref, xl_ref, xr_ref):
    f = (g_ref[...] * (h_ref[...] - mu_ref[...])) / den_ref[...] + be_ref[...]
    f_ref[...] = f
    xl_ref[...] = jnp.dot(f, wl_ref[...], preferred_element_type=jnp.float32)
    xr_ref[...] = jnp.dot(f, wr_ref[...], preferred_element_type=jnp.float32)


def _proj(h, gcol, mucol, dencol, becol, wl, wr):
    blk = B // 4
    return pl.pallas_call(
        _proj_body,
        grid=(4,),
        in_specs=[
            pl.BlockSpec((blk, D), lambda i: (i, 0)),
            pl.BlockSpec((1, D), lambda i: (0, 0)),
            pl.BlockSpec((1, D), lambda i: (0, 0)),
            pl.BlockSpec((1, D), lambda i: (0, 0)),
            pl.BlockSpec((1, D), lambda i: (0, 0)),
            pl.BlockSpec((D, DH), lambda i: (0, 0)),
            pl.BlockSpec((D, DH), lambda i: (0, 0)),
        ],
        out_specs=[
            pl.BlockSpec((blk, D), lambda i: (i, 0)),
            pl.BlockSpec((blk, DH), lambda i: (i, 0)),
            pl.BlockSpec((blk, DH), lambda i: (i, 0)),
        ],
        out_shape=[
            jax.ShapeDtypeStruct((B, D), jnp.float32),
            jax.ShapeDtypeStruct((B, DH), jnp.float32),
            jax.ShapeDtypeStruct((B, DH), jnp.float32),
        ],
    )(h, gcol, mucol, dencol, becol, wl, wr)


def _knn_body(f_ref, ft_ref, idx_ref, sq_scr, *, row0):
    i = pl.program_id(0)

    @pl.when(i == 0)
    def _():
        ft = ft_ref[...]
        sq_scr[...] = jnp.sum(ft * ft, axis=0, keepdims=True)

    fb = f_ref[...]
    sq_b = jnp.sum(fb * fb, axis=1, keepdims=True)
    t = lax.dot_general(fb, ft_ref[...], (((1,), (0,)), ((), ())),
                        preferred_element_type=jnp.float32)
    d2 = (sq_b + sq_scr[...]) - 2.0 * t
    rblk = fb.shape[0]
    cols = lax.broadcasted_iota(jnp.int32, (rblk, B), 1)
    rows_g = row0 + i * rblk + lax.broadcasted_iota(jnp.int32, (rblk, B), 0)
    neg = jnp.where(cols == rows_g, NEG, -d2)
    for k in range(K):
        m = jnp.max(neg, axis=1, keepdims=True)
        cand = jnp.where(neg >= m, cols, B)
        amin = jnp.min(cand, axis=1, keepdims=True)
        idx_ref[:, k:k + 1] = amin
        neg = jnp.where(cols == amin, NEG, neg)


def _knn(feat, featT, off, nrows):
    blk = 256
    ob = off // blk
    return pl.pallas_call(
        functools.partial(_knn_body, row0=off),
        grid=(nrows // blk,),
        in_specs=[
            pl.BlockSpec((blk, D), lambda i: (i + ob, 0)),
            pl.BlockSpec((D, B), lambda i: (0, 0)),
        ],
        out_specs=pl.BlockSpec((blk, K), lambda i: (i, 0)),
        out_shape=jax.ShapeDtypeStruct((nrows, K), jnp.int32),
        scratch_shapes=[pltpu.VMEM((1, B), jnp.float32)],
    )(feat, featT)


def _conv2_body(go_ref, gb_ref, m_ref, b_ref, t_ref, s_ref):
    g = go_ref[...] + gb_ref[...]
    t = jnp.dot(g, m_ref[...], preferred_element_type=jnp.float32)
    t = t + b_ref[...]
    t_ref[...] = t
    s_ref[0:1, :] = jnp.sum(t, axis=0, keepdims=True)
    s_ref[1:2, :] = jnp.sum(t * t, axis=0, keepdims=True)


def _conv2(go, gbrow, m2, bb2):
    return pl.pallas_call(
        _conv2_body,
        grid=(1,),
        in_specs=[
            pl.BlockSpec((B, DH), lambda i: (0, 0)),
            pl.BlockSpec((1, DH), lambda i: (0, 0)),
            pl.BlockSpec((DH, D), lambda i: (0, 0)),
            pl.BlockSpec((1, D), lambda i: (0, 0)),
        ],
        out_specs=[
            pl.BlockSpec((B, D), lambda i: (0, 0)),
            pl.BlockSpec((2, D), lambda i: (0, 0)),
        ],
        out_shape=[
            jax.ShapeDtypeStruct((B, D), jnp.float32),
            jax.ShapeDtypeStruct((2, D), jnp.float32),
        ],
    )(go, gbrow, m2, bb2)


def _bnres_body(t_ref, g_ref, mu_ref, den_ref, be_ref, x_ref, y_ref):
    y = (g_ref[...] * (t_ref[...] - mu_ref[...])) / den_ref[...] + be_ref[...]
    y_ref[...] = y + x_ref[...]


def _bnres(t2d, gcol, mucol, dencol, becol, x2d):
    return pl.pallas_call(
        _bnres_body,
        grid=(1,),
        in_specs=[
            pl.BlockSpec((B, D), lambda i: (0, 0)),
            pl.BlockSpec((1, D), lambda i: (0, 0)),
            pl.BlockSpec((1, D), lambda i: (0, 0)),
            pl.BlockSpec((1, D), lambda i: (0, 0)),
            pl.BlockSpec((1, D), lambda i: (0, 0)),
            pl.BlockSpec((B, D), lambda i: (0, 0)),
        ],
        out_specs=pl.BlockSpec((B, D), lambda i: (0, 0)),
        out_shape=jax.ShapeDtypeStruct((B, D), jnp.float32),
    )(t2d, gcol, mucol, dencol, becol, x2d)


def _bn_cols(stats, g, be, nch):
    n = jnp.float32(B * P)
    cs = stats[0].reshape(nch, P).sum(axis=1)
    css = stats[1].reshape(nch, P).sum(axis=1)
    mu = cs / n
    var = css / n - mu * mu
    den = jnp.sqrt(var + 1e-5)
    rep = lambda v: jnp.repeat(v, P).reshape(1, nch * P)
    return rep(g), rep(mu), rep(den), rep(be)


_GATHER_DNUMS = lax.GatherDimensionNumbers(
    offset_dims=(), collapsed_slice_dims=(0,), start_index_map=(0,))


def _take(v, idx):
    return lax.gather(v, idx[:, None], _GATHER_DNUMS, (1,),
                      mode=lax.GatherScatterMode.PROMISE_IN_BOUNDS)


def _bsum(v, lane):
    for s in (8, 4, 2, 1):
        v = v + _take(v, lane ^ s)
    return v


def _bmax(v, lane):
    for s in (8, 4, 2, 1):
        v = jnp.maximum(v, _take(v, lane ^ s))
    return v


def _sc_agg_body(xl_hbm, xr_hbm, idx_hbm, att_hbm, go_hbm,
                 idxs_v, att_v, rows_a, rows_b, xr_a, xr_b, out_a, out_b,
                 sem_a, sem_b, osem_a, osem_b, *, row_off, npw):
    wid = lax.axis_index("s") * 2 + lax.axis_index("c")
    base = wid * npw
    pltpu.sync_copy(idx_hbm.at[pl.ds(base * K, npw * K)], idxs_v)
    pltpu.sync_copy(att_hbm.at[:], att_v)
    lane = lax.iota(jnp.int32, 16)

    def launch(n, rows_v, xr_v, sem):
        node = row_off + base + n
        pltpu.make_async_copy(xl_hbm.at[idxs_v.at[pl.ds(n * K, K)]],
                              rows_v.at[pl.ds(0, K)], sem).start()
        pltpu.make_async_copy(xl_hbm.at[pl.ds(node, 1)],
                              rows_v.at[pl.ds(K, 1)], sem).start()
        pltpu.make_async_copy(xr_hbm.at[pl.ds(node, 1)], xr_v, sem).start()

    def wait_inputs(n, rows_v, xr_v, sem):
        pltpu.make_async_copy(xl_hbm.at[idxs_v.at[pl.ds(n * K, K)]],
                              rows_v.at[pl.ds(0, K)], sem).wait()
        pltpu.make_async_copy(xl_hbm.at[pl.ds(base, 1)],
                              rows_v.at[pl.ds(K, 1)], sem).wait()
        pltpu.make_async_copy(xr_hbm.at[pl.ds(base, 1)], xr_v, sem).wait()

    def compute(g, n, rows_v, xr_v, out_v, sem, osem):
        wait_inputs(n, rows_v, xr_v, sem)

        @pl.when(g > 0)
        def _():
            pltpu.make_async_copy(out_v, go_hbm.at[pl.ds(base, 1)],
                                  osem).wait()

        for h in range(HEADS):
            def c_body(c, accs, h=h):
                off = h * D + c * 16
                x = xr_v[0, pl.ds(off, 16)]
                a = att_v[pl.ds(off, 16)]
                new = []
                for kk in range(K + 1):
                    e = rows_v[kk, pl.ds(off, 16)] + x
                    e = jnp.maximum(e, 0.2 * e)
                    new.append(accs[kk] + e * a)
                return tuple(new)

            zeros = tuple(jnp.zeros((16,), jnp.float32)
                          for _ in range(K + 1))
            accs = lax.fori_loop(0, P, c_body, zeros)
            lg = [_bsum(v, lane) for v in accs]
            m = lg[0]
            for v in lg[1:]:
                m = jnp.maximum(m, v)
            es = [jnp.exp(v - m) for v in lg]
            den = es[0]
            for v in es[1:]:
                den = den + v
            den = den + 1e-16
            alph = [v / den for v in es]

            def w_body(c, c2, h=h, alph=alph):
                off = h * D + c * 16
                acc = alph[0] * rows_v[0, pl.ds(off, 16)]
                for kk in range(1, K + 1):
                    acc = acc + alph[kk] * rows_v[kk, pl.ds(off, 16)]
                out_v[0, pl.ds(off, 16)] = acc
                return c2

            lax.fori_loop(0, P, w_body, 0)

        node = base + n
        pltpu.make_async_copy(out_v, go_hbm.at[pl.ds(node, 1)], osem).start()

    launch(0, rows_a, xr_a, sem_a)
    launch(1, rows_b, xr_b, sem_b)

    def g_body(g, carry):
        compute(g, 2 * g, rows_a, xr_a, out_a, sem_a, osem_a)

        @pl.when(g < npw // 2 - 1)
        def _():
            launch(2 * g + 2, rows_a, xr_a, sem_a)

        compute(g, 2 * g + 1, rows_b, xr_b, out_b, sem_b, osem_b)

        @pl.when(g < npw // 2 - 1)
        def _():
            launch(2 * g + 3, rows_b, xr_b, sem_b)

        return carry

    lax.fori_loop(0, npw // 2, g_body, 0)
    pltpu.make_async_copy(out_a, go_hbm.at[pl.ds(base, 1)], osem_a).wait()
    pltpu.make_async_copy(out_b, go_hbm.at[pl.ds(base, 1)], osem_b).wait()


def _sc_agg(xl, xr, idx, attf, row_off, nrows):
    npw = nrows // NWORK
    mesh = plsc.VectorSubcoreMesh(core_axis_name="c", subcore_axis_name="s")
    return pl.kernel(
        functools.partial(_sc_agg_body, row_off=row_off, npw=npw),
        out_type=jax.ShapeDtypeStruct((nrows, DH), jnp.float32),
        mesh=mesh,
        scratch_types=[
            pltpu.VMEM((npw * K,), jnp.int32),
            pltpu.VMEM((DH,), jnp.float32),
            pltpu.VMEM((K + 1, DH), jnp.float32),
            pltpu.VMEM((K + 1, DH), jnp.float32),
            pltpu.VMEM((1, DH), jnp.float32),
            pltpu.VMEM((1, DH), jnp.float32),
            pltpu.VMEM((1, DH), jnp.float32),
            pltpu.VMEM((1, DH), jnp.float32),
            pltpu.SemaphoreType.DMA,
            pltpu.SemaphoreType.DMA,
            pltpu.SemaphoreType.DMA,
            pltpu.SemaphoreType.DMA,
        ],
    )(xl, xr, idx.reshape(nrows * K), attf)


def kernel(x, NUM_NEIGHBORS, EDGE_METHOD, W1, b1, g1, be1, Wl, Wr, att, gb,
           W2, b2, g2, be2):
    x2d = x.reshape(B, D)
    eye = jnp.eye(P, dtype=jnp.float32)
    m1 = jnp.kron(W1.T, eye)
    bb1 = jnp.repeat(b1, P).reshape(1, D)
    m2 = jnp.kron(W2.T, eye)
    bb2 = jnp.repeat(b2, P).reshape(1, D)
    attf = att.reshape(DH)
    gbrow = gb.reshape(1, DH)

    h, s1 = _conv1(x2d, m1, bb1)
    g1c, mu1c, den1c, be1c = _bn_cols(s1, g1, be1, C)
    feat, xl, xr = _proj(h, g1c, mu1c, den1c, be1c, Wl, Wr)
    featT = feat.T
    nsplit = 4
    step = B // nsplit
    idxs = [_knn(feat, featT, s * step, step) for s in range(nsplit)]
    gos = [_sc_agg(xl, xr, idxs[s], attf, s * step, step)
           for s in range(nsplit)]
    go = jnp.concatenate(gos, axis=0)
    t2d, s2 = _conv2(go, gbrow, m2, bb2)
    g2c, mu2c, den2c, be2c = _bn_cols(s2, g2, be2, C)
    y2d = _bnres(t2d, g2c, mu2c, den2c, be2c, x2d)
    return y2d.reshape(B, C, 4, 4)
